# speedup vs baseline: 1.5081x; 1.5081x over previous
import functools

import jax
import jax.numpy as jnp
from jax import lax
from jax.experimental import pallas as pl
from jax.experimental.pallas import tpu as pltpu
from jax.experimental.pallas import tpu_sc as plsc

MAX_ROWS = 8192
D_MODEL = 1024

_info = plsc.get_sparse_core_info()
_NC, _NS = _info.num_cores, _info.num_subcores
_NW = _NC * _NS
_ROWS_PER_W = MAX_ROWS // _NW
_CHUNK = 64
_NCHUNK = _ROWS_PER_W // _CHUNK

_mesh = plsc.VectorSubcoreMesh(core_axis_name="c", subcore_axis_name="s")


@functools.partial(
    pl.kernel,
    mesh=_mesh,
    out_type=jax.ShapeDtypeStruct((MAX_ROWS, D_MODEL), jnp.float32),
    scratch_types=[
        pltpu.VMEM((_NCHUNK, _CHUNK), jnp.int32),
        pltpu.VMEM((_CHUNK, D_MODEL), jnp.float32),
        pltpu.SemaphoreType.DMA,
    ],
)
def _gather_kernel(table_hbm, idx_hbm, out_hbm, idx_v, rows_v, sem):
    wid = lax.axis_index("s") * _NC + lax.axis_index("c")
    base = wid * _ROWS_PER_W
    pltpu.sync_copy(idx_hbm.at[wid], idx_v)
    for i in range(_NCHUNK):
        pltpu.async_copy(table_hbm.at[idx_v.at[i]], rows_v, sem).wait()
        pltpu.sync_copy(rows_v, out_hbm.at[pl.ds(base + i * _CHUNK, _CHUNK)])


def kernel(seq_len, embedding_weight):
    idx = jnp.arange(MAX_ROWS, dtype=jnp.int32)
    idx = jnp.where(idx < seq_len, idx, 0).reshape(_NW, _NCHUNK, _CHUNK)
    return _gather_kernel(embedding_weight, idx)


# speedup vs baseline: 1.5625x; 1.0360x over previous
import functools

import jax
import jax.numpy as jnp
from jax import lax
from jax.experimental import pallas as pl
from jax.experimental.pallas import tpu as pltpu
from jax.experimental.pallas import tpu_sc as plsc

MAX_ROWS = 8192
D_MODEL = 1024

_info = plsc.get_sparse_core_info()
_NC, _NS = _info.num_cores, _info.num_subcores
_NW = _NC * _NS
_ROWS_PER_W = MAX_ROWS // _NW
_CHUNK = 32
_NCHUNK = _ROWS_PER_W // _CHUNK

_mesh = plsc.VectorSubcoreMesh(core_axis_name="c", subcore_axis_name="s")


@functools.partial(
    pl.kernel,
    mesh=_mesh,
    out_type=jax.ShapeDtypeStruct((MAX_ROWS, D_MODEL), jnp.float32),
    scratch_types=[
        pltpu.VMEM((_NCHUNK, _CHUNK), jnp.int32),
        pltpu.VMEM((2, _CHUNK, D_MODEL), jnp.float32),
        pltpu.SemaphoreType.DMA,
        pltpu.SemaphoreType.DMA,
    ],
)
def _gather_kernel(table_hbm, idx_hbm, out_hbm, idx_v, rows_v, sem_in, sem_out):
    wid = lax.axis_index("s") * _NC + lax.axis_index("c")
    base = wid * _ROWS_PER_W
    pltpu.sync_copy(idx_hbm.at[wid], idx_v)

    def gather(i, b):
        return pltpu.async_copy(table_hbm.at[idx_v.at[i]], rows_v.at[b], sem_in)

    def put(i, b):
        return pltpu.async_copy(
            rows_v.at[b], out_hbm.at[pl.ds(base + i * _CHUNK, _CHUNK)], sem_out)

    ins = [None] * _NCHUNK
    outs = [None] * _NCHUNK
    ins[0] = gather(0, 0)
    for i in range(_NCHUNK):
        b = i & 1
        if i >= 1:
            outs[i - 1].wait()
        if i + 1 < _NCHUNK:
            ins[i + 1] = gather(i + 1, 1 - b)
        ins[i].wait()
        outs[i] = put(i, b)
    outs[_NCHUNK - 1].wait()


def kernel(seq_len, embedding_weight):
    idx = jnp.arange(MAX_ROWS, dtype=jnp.int32)
    idx = jnp.where(idx < seq_len, idx, 0).reshape(_NW, _NCHUNK, _CHUNK)
    return _gather_kernel(embedding_weight, idx)
